# baseline (speedup 1.0000x reference)
import functools

import jax
import jax.numpy as jnp
from jax import lax
from jax.experimental import pallas as pl
from jax.experimental.pallas import tpu as pltpu
from jax.experimental.pallas import tpu_sc as plsc

N = 10000
E = 320000
D = 128
DE = 16
G = 100

NC = 2
NS = 16
NW = NC * NS
EPW = E // NW
CH = 80
NCHUNK = EPW // CH

_f32 = jnp.float32


def _mm_body(h_ref, w_ref, oa_ref, ob_ref):
    ab = jnp.dot(h_ref[...], w_ref[...], preferred_element_type=_f32)
    oa_ref[...] = ab[:, :D]
    ob_ref[...] = ab[:, D:]


def _node_proj(h, w_ab):
    blk = 2000
    return pl.pallas_call(
        _mm_body,
        grid=(N // blk,),
        in_specs=[
            pl.BlockSpec((blk, D), lambda i: (i, 0)),
            pl.BlockSpec((D, 2 * D), lambda i: (0, 0)),
        ],
        out_specs=[
            pl.BlockSpec((blk, D), lambda i: (i, 0)),
            pl.BlockSpec((blk, D), lambda i: (i, 0)),
        ],
        out_shape=[
            jax.ShapeDtypeStruct((N, D), _f32),
            jax.ShapeDtypeStruct((N, D), _f32),
        ],
    )(h, w_ab)


def _make_gather_body(epw, nchunk):
  def _sc_gather_body(ta_hbm, tb_hbm, row_hbm, col_hbm, wr_hbm, xt_hbm,
                      m_hbm,
                      idxr1, idxc1, ta, tb, mv, rad_all, wr_v, xtab_v,
                      gsem, wsem):
    EPW, NCHUNK = epw, nchunk
    cid = lax.axis_index("c")
    sid = lax.axis_index("s")
    wid = sid * NC + cid
    pltpu.sync_copy(wr_hbm, wr_v)
    pltpu.sync_copy(xt_hbm, xtab_v)
    pltpu.sync_copy(row_hbm.at[wid], idxr1)
    pltpu.sync_copy(col_hbm.at[wid], idxc1)
    ws = [wr_v[pl.ds(16 * j, 16)] for j in range(D // 16)]

    def fetch(b, i):
        pltpu.async_copy(ta_hbm.at[idxr1.at[pl.ds(i * CH, CH)]], ta[b],
                         gsem[b])
        pltpu.async_copy(tb_hbm.at[idxc1.at[pl.ds(i * CH, CH)]], tb[b],
                         gsem[b])

    def drain_gather(b, i):
        pltpu.make_async_copy(ta_hbm.at[idxr1.at[pl.ds(i * CH, CH)]], ta[b],
                              gsem[b]).wait()
        pltpu.make_async_copy(tb_hbm.at[idxc1.at[pl.ds(i * CH, CH)]], tb[b],
                              gsem[b]).wait()

    def process(b, i):
        base = wid * EPW + i * CH
        for g in range(CH // 16):
            iv_r = idxr1[pl.ds(i * CH + 16 * g, 16)]
            iv_c = idxc1[pl.ds(i * CH + 16 * g, 16)]
            rad16 = jnp.zeros((16,), _f32)
            for c in range(3):
                off = jnp.int32(c * N)
                dxy = (plsc.load_gather(xtab_v, [iv_r + off])
                       - plsc.load_gather(xtab_v, [iv_c + off]))
                rad16 = rad16 + dxy * dxy
            rad_all[pl.ds(i * CH + 16 * g, 16)] = rad16
        drain_gather(b, i)

        @plsc.parallel_loop(0, CH, 1, unroll=4)
        def edge(e):
            rad = plsc.load_gather(
                rad_all, [jnp.full((16,), i * CH + e, jnp.int32)])
            for j in range(D // 16):
                mv[b][e, pl.ds(16 * j, 16)] = (
                    ta[b][e, pl.ds(16 * j, 16)]
                    + tb[b][e, pl.ds(16 * j, 16)]
                    + rad * ws[j])
        pltpu.async_copy(mv[b], m_hbm.at[pl.ds(base, CH)], wsem[b])

    def drain_write(b, i):
        base = wid * EPW + i * CH
        pltpu.make_async_copy(mv[b], m_hbm.at[pl.ds(base, CH)], wsem[b]).wait()

    fetch(0, 0)

    def pair(t, carry):
        i0 = 2 * t
        fetch(1, i0 + 1)

        @pl.when(t > 0)
        def _():
            drain_write(0, i0 - 2)
        process(0, i0)

        @pl.when(i0 + 2 < NCHUNK)
        def _():
            fetch(0, i0 + 2)

        @pl.when(t > 0)
        def _():
            drain_write(1, i0 - 1)
        process(1, i0 + 1)
        return carry

    lax.fori_loop(0, NCHUNK // 2, pair, 0)
    if NCHUNK % 2 == 1:
        drain_write(0, NCHUNK - 3)
        process(0, NCHUNK - 1)
        drain_write(1, NCHUNK - 2)
        drain_write(0, NCHUNK - 1)
    else:
        drain_write(0, NCHUNK - 2)
        drain_write(1, NCHUNK - 1)
  return _sc_gather_body


def _sc_gather(ta, tb, row2, col2, wr, xt, epw):
    mesh = plsc.VectorSubcoreMesh(core_axis_name="c", subcore_axis_name="s")
    f = pl.kernel(
        _make_gather_body(epw, epw // CH),
        out_type=jax.ShapeDtypeStruct((NW * epw, D), _f32),
        mesh=mesh,
        compiler_params=pltpu.CompilerParams(needs_layout_passes=False),
        scratch_types=[
            pltpu.VMEM((epw,), jnp.int32),
            pltpu.VMEM((epw,), jnp.int32),
            [pltpu.VMEM((CH, D), _f32)] * 2,
            [pltpu.VMEM((CH, D), _f32)] * 2,
            [pltpu.VMEM((CH, D), _f32)] * 2,
            pltpu.VMEM((epw,), _f32),
            pltpu.VMEM((D,), _f32),
            pltpu.VMEM((3 * N,), _f32),
            [pltpu.SemaphoreType.DMA] * 2,
            [pltpu.SemaphoreType.DMA] * 2,
        ],
    )
    return f(ta, tb, row2, col2, wr, xt)


def _edge_mlp_body(m_ref, eat_ref, wa_ref, be1_ref, w2_ref, be2_ref, o_ref):
    c = lax.dot_general(eat_ref[...], wa_ref[...],
                        (((0,), (0,)), ((), ())),
                        preferred_element_type=_f32)
    pre = m_ref[...] + c + be1_ref[...]
    mm = pre * jax.nn.sigmoid(pre)
    ef = jnp.dot(mm, w2_ref[...], preferred_element_type=_f32) + be2_ref[...]
    o_ref[...] = ef * jax.nn.sigmoid(ef)


def _edge_mlp(m, ea_t, w_attr, be1, we2, be2, blk_off):
    blk = 2560
    es = m.shape[0]
    return pl.pallas_call(
        _edge_mlp_body,
        grid=(es // blk,),
        in_specs=[
            pl.BlockSpec((blk, D), lambda i: (i, 0)),
            pl.BlockSpec((DE, blk), lambda i: (0, i + blk_off)),
            pl.BlockSpec((DE, D), lambda i: (0, 0)),
            pl.BlockSpec((1, D), lambda i: (0, 0)),
            pl.BlockSpec((D, D), lambda i: (0, 0)),
            pl.BlockSpec((1, D), lambda i: (0, 0)),
        ],
        out_specs=pl.BlockSpec((blk, D), lambda i: (i, 0)),
        out_shape=jax.ShapeDtypeStruct((es, D), _f32),
    )(m, ea_t, w_attr, be1.reshape(1, D), we2, be2.reshape(1, D))


_ZR = 1000


def _make_scatter_body(epw, nchunk):
  def _sc_scatter_body(ef_hbm, row_hbm, zero_hbm, agg_hbm,
                       idx, efv, agg_sh, lsem):
    EPW, NCHUNK = epw, nchunk
    cid = lax.axis_index("c")
    sid = lax.axis_index("s")
    wid = sid * NC + cid

    @pl.when(sid < N // _ZR)
    def _zero():
        pltpu.sync_copy(zero_hbm, agg_sh.at[pl.ds(sid * _ZR, _ZR)])

    plsc.subcore_barrier()

    def fetch(b, i):
        base = wid * EPW + i * CH
        pltpu.sync_copy(row_hbm.at[wid, i], idx[b])
        pltpu.async_copy(ef_hbm.at[pl.ds(base, CH)], efv[b], lsem[b])

    def scat(b, i):
        base = wid * EPW + i * CH
        pltpu.make_async_copy(ef_hbm.at[pl.ds(base, CH)], efv[b],
                              lsem[b]).wait()
        pltpu.sync_copy(efv[b], agg_sh.at[idx[b]], add=True)

    NB = 4
    for b in range(NB):
        fetch(b, b)

    def quad(t, carry):
        i0 = NB * t
        for b in range(NB):
            scat(b, i0 + b)

            @pl.when(i0 + b + NB < NCHUNK)
            def _():
                fetch(b, i0 + b + NB)
        return carry

    lax.fori_loop(0, NCHUNK // NB, quad, 0)
    for r in range(NB * (NCHUNK // NB), NCHUNK):
        scat(r % NB, r)
    plsc.subcore_barrier()

    @pl.when(sid < N // _ZR)
    def _writeback():
        pltpu.sync_copy(agg_sh.at[pl.ds(sid * _ZR, _ZR)],
                        agg_hbm.at[pl.ds(cid * N + sid * _ZR, _ZR)])
  return _sc_scatter_body


def _sc_scatter(ef, row3, zeros_blk, epw):
    mesh = plsc.VectorSubcoreMesh(core_axis_name="c", subcore_axis_name="s")
    f = pl.kernel(
        _make_scatter_body(epw, epw // CH),
        out_type=jax.ShapeDtypeStruct((2 * N, D), _f32),
        mesh=mesh,
        compiler_params=pltpu.CompilerParams(needs_layout_passes=False),
        scratch_types=[
            [pltpu.VMEM((CH,), jnp.int32)] * 4,
            [pltpu.VMEM((CH, D), _f32)] * 4,
            pltpu.VMEM_SHARED((N, D), _f32),
            [pltpu.SemaphoreType.DMA] * 4,
        ],
    )
    return f(ef, row3, zeros_blk)


def _node_body(*refs):
    nagg = len(refs) - 10
    (h_ref, *agg_refs), (b_ref, w1h_ref, w1a_ref, bn1_ref, w2_ref, bn2_ref,
                         o_ref, sums, cnts) = refs[:1 + nagg], refs[1 + nagg:]
    i = pl.program_id(0)

    @pl.when(i == 0)
    def _init():
        sums[...] = jnp.zeros_like(sums)
        cnts[...] = jnp.zeros_like(cnts)

    hb = h_ref[...]
    agg = agg_refs[0][...]
    for a in agg_refs[1:]:
        agg = agg + a[...]
    t = (jnp.dot(hb, w1h_ref[...], preferred_element_type=_f32)
         + jnp.dot(agg, w1a_ref[...], preferred_element_type=_f32)
         + bn1_ref[...])
    t = t * jax.nn.sigmoid(t)
    out = jnp.dot(t, w2_ref[...], preferred_element_type=_f32) + bn2_ref[...]
    hn = hb + out
    R = hb.shape[0]
    oh = jnp.equal(lax.broadcasted_iota(jnp.int32, (128, R), 0),
                   b_ref[0]).astype(_f32)
    sums[...] += jnp.dot(oh, hn, preferred_element_type=_f32)
    cnts[...] += jnp.dot(oh, jnp.ones((R, D), _f32),
                         preferred_element_type=_f32)

    @pl.when(i == pl.num_programs(0) - 1)
    def _fin():
        o_ref[...] = sums[...] / jnp.maximum(cnts[...], 1.0)


def _node_pool(h, aggs, batch3, wn1h, wn1a, bn1, wn2, bn2):
    blk = 1000
    nb = N // blk
    agg_specs = []
    agg_args = []
    for a in aggs:
        agg_specs += [pl.BlockSpec((blk, D), lambda i: (i, 0)),
                      pl.BlockSpec((blk, D), lambda i: (i + nb, 0))]
        agg_args += [a, a]
    return pl.pallas_call(
        _node_body,
        grid=(nb,),
        in_specs=[pl.BlockSpec((blk, D), lambda i: (i, 0))] + agg_specs + [
            pl.BlockSpec((1, 1, blk), lambda i: (i, 0, 0)),
            pl.BlockSpec((D, D), lambda i: (0, 0)),
            pl.BlockSpec((D, D), lambda i: (0, 0)),
            pl.BlockSpec((1, D), lambda i: (0, 0)),
            pl.BlockSpec((D, D), lambda i: (0, 0)),
            pl.BlockSpec((1, D), lambda i: (0, 0)),
        ],
        out_specs=pl.BlockSpec((128, D), lambda i: (0, 0)),
        out_shape=jax.ShapeDtypeStruct((128, D), _f32),
        scratch_shapes=[
            pltpu.VMEM((128, D), _f32),
            pltpu.VMEM((128, D), _f32),
        ],
    )(h, *agg_args, batch3, wn1h, wn1a,
      bn1.reshape(1, D), wn2, bn2.reshape(1, D))


_SLABS = (107520, 107520, 104960)


def kernel(h, edge_index, x, edge_attr, batch,
           We1, be1, We2, be2, Wn1, bn1, Wn2, bn2):
    row = edge_index[0]
    col = edge_index[1]

    w_ab = jnp.concatenate([We1[:D], We1[D:2 * D]], axis=1)
    ta, tb = _node_proj(h, w_ab)
    xt = x.T.reshape(3 * N)

    wr = We1[2 * D]
    w_attr = We1[2 * D + 1:]
    ea_t = edge_attr.T
    zeros_blk = jnp.zeros((_ZR, D), _f32)

    aggs = []
    off = 0
    for es in _SLABS:
        epw = es // NW
        row_s = lax.slice(row, (off,), (off + es,))
        col_s = lax.slice(col, (off,), (off + es,))
        m_s = _sc_gather(ta, tb, row_s.reshape(NW, epw),
                         col_s.reshape(NW, epw), wr, xt, epw)
        ef_s = _edge_mlp(m_s, ea_t, w_attr, be1, We2, be2, off // 2560)
        aggs.append(_sc_scatter(ef_s, row_s.reshape(NW, epw // CH, CH),
                                zeros_blk, epw))
        off += es

    batch3 = batch.reshape(N // 1000, 1, 1000)
    p_full = _node_pool(h, aggs, batch3, Wn1[:D], Wn1[D:], bn1, Wn2, bn2)
    return p_full[:G]


# speedup vs baseline: 1.0763x; 1.0763x over previous
import functools

import jax
import jax.numpy as jnp
from jax import lax
from jax.experimental import pallas as pl
from jax.experimental.pallas import tpu as pltpu
from jax.experimental.pallas import tpu_sc as plsc

N = 10000
E = 320000
D = 128
DE = 16
G = 100

NC = 2
NS = 16
NW = NC * NS
EPW = E // NW
CH = 80
NCHUNK = EPW // CH

_f32 = jnp.float32


def _mm_body(h_ref, w_ref, oa_ref, ob_ref):
    ab = jnp.dot(h_ref[...], w_ref[...], preferred_element_type=_f32)
    oa_ref[...] = ab[:, :D]
    ob_ref[...] = ab[:, D:]


def _node_proj(h, w_ab):
    blk = 2000
    return pl.pallas_call(
        _mm_body,
        grid=(N // blk,),
        in_specs=[
            pl.BlockSpec((blk, D), lambda i: (i, 0)),
            pl.BlockSpec((D, 2 * D), lambda i: (0, 0)),
        ],
        out_specs=[
            pl.BlockSpec((blk, D), lambda i: (i, 0)),
            pl.BlockSpec((blk, D), lambda i: (i, 0)),
        ],
        out_shape=[
            jax.ShapeDtypeStruct((N, D), _f32),
            jax.ShapeDtypeStruct((N, D), _f32),
        ],
    )(h, w_ab)


def _make_gather_body(epw, nchunk):
  def _sc_gather_body(ta_hbm, tb_hbm, row_hbm, col_hbm, wr_hbm, xt_hbm,
                      m_hbm,
                      idxr1, idxc1, ta, tb, mv, rad_all, wr_v, xtab_v,
                      gsem, wsem):
    EPW, NCHUNK = epw, nchunk
    cid = lax.axis_index("c")
    sid = lax.axis_index("s")
    wid = sid * NC + cid
    pltpu.sync_copy(wr_hbm, wr_v)
    pltpu.sync_copy(xt_hbm, xtab_v)
    pltpu.sync_copy(row_hbm.at[wid], idxr1)
    pltpu.sync_copy(col_hbm.at[wid], idxc1)
    ws = [wr_v[pl.ds(16 * j, 16)] for j in range(D // 16)]

    def fetch(b, i):
        pltpu.async_copy(ta_hbm.at[idxr1.at[pl.ds(i * CH, CH)]], ta[b],
                         gsem[b])
        pltpu.async_copy(tb_hbm.at[idxc1.at[pl.ds(i * CH, CH)]], tb[b],
                         gsem[b])

    def drain_gather(b, i):
        pltpu.make_async_copy(ta_hbm.at[idxr1.at[pl.ds(i * CH, CH)]], ta[b],
                              gsem[b]).wait()
        pltpu.make_async_copy(tb_hbm.at[idxc1.at[pl.ds(i * CH, CH)]], tb[b],
                              gsem[b]).wait()

    def process(b, i):
        base = wid * EPW + i * CH
        for g in range(CH // 16):
            iv_r = idxr1[pl.ds(i * CH + 16 * g, 16)]
            iv_c = idxc1[pl.ds(i * CH + 16 * g, 16)]
            rad16 = jnp.zeros((16,), _f32)
            for c in range(3):
                off = jnp.int32(c * N)
                dxy = (plsc.load_gather(xtab_v, [iv_r + off])
                       - plsc.load_gather(xtab_v, [iv_c + off]))
                rad16 = rad16 + dxy * dxy
            rad_all[pl.ds(i * CH + 16 * g, 16)] = rad16
        drain_gather(b, i)

        @plsc.parallel_loop(0, CH, 1, unroll=4)
        def edge(e):
            rad = plsc.load_gather(
                rad_all, [jnp.full((16,), i * CH + e, jnp.int32)])
            for j in range(D // 16):
                mv[b][e, pl.ds(16 * j, 16)] = (
                    ta[b][e, pl.ds(16 * j, 16)]
                    + tb[b][e, pl.ds(16 * j, 16)]
                    + rad * ws[j])
        pltpu.async_copy(mv[b], m_hbm.at[pl.ds(base, CH)], wsem[b])

    def drain_write(b, i):
        base = wid * EPW + i * CH
        pltpu.make_async_copy(mv[b], m_hbm.at[pl.ds(base, CH)], wsem[b]).wait()

    fetch(0, 0)

    def pair(t, carry):
        i0 = 2 * t
        fetch(1, i0 + 1)

        @pl.when(t > 0)
        def _():
            drain_write(0, i0 - 2)
        process(0, i0)

        @pl.when(i0 + 2 < NCHUNK)
        def _():
            fetch(0, i0 + 2)

        @pl.when(t > 0)
        def _():
            drain_write(1, i0 - 1)
        process(1, i0 + 1)
        return carry

    lax.fori_loop(0, NCHUNK // 2, pair, 0)
    if NCHUNK % 2 == 1:
        drain_write(0, NCHUNK - 3)
        process(0, NCHUNK - 1)
        drain_write(1, NCHUNK - 2)
        drain_write(0, NCHUNK - 1)
    else:
        drain_write(0, NCHUNK - 2)
        drain_write(1, NCHUNK - 1)
  return _sc_gather_body


def _sc_gather(ta, tb, row2, col2, wr, xt, epw):
    mesh = plsc.VectorSubcoreMesh(core_axis_name="c", subcore_axis_name="s")
    f = pl.kernel(
        _make_gather_body(epw, epw // CH),
        out_type=jax.ShapeDtypeStruct((NW * epw, D), _f32),
        mesh=mesh,
        compiler_params=pltpu.CompilerParams(needs_layout_passes=False),
        scratch_types=[
            pltpu.VMEM((epw,), jnp.int32),
            pltpu.VMEM((epw,), jnp.int32),
            [pltpu.VMEM((CH, D), _f32)] * 2,
            [pltpu.VMEM((CH, D), _f32)] * 2,
            [pltpu.VMEM((CH, D), _f32)] * 2,
            pltpu.VMEM((epw,), _f32),
            pltpu.VMEM((D,), _f32),
            pltpu.VMEM((3 * N,), _f32),
            [pltpu.SemaphoreType.DMA] * 2,
            [pltpu.SemaphoreType.DMA] * 2,
        ],
    )
    return f(ta, tb, row2, col2, wr, xt)


def _edge_mlp_body(m_ref, eat_ref, wa_ref, be1_ref, w2_ref, be2_ref, o_ref):
    c = lax.dot_general(eat_ref[...], wa_ref[...],
                        (((0,), (0,)), ((), ())),
                        preferred_element_type=_f32)
    pre = m_ref[...] + c + be1_ref[...]
    mm = pre * jax.nn.sigmoid(pre)
    ef = jnp.dot(mm, w2_ref[...], preferred_element_type=_f32) + be2_ref[...]
    o_ref[...] = ef * jax.nn.sigmoid(ef)


def _edge_mlp(m, ea_t, w_attr, be1, we2, be2, blk_off):
    blk = 2560
    es = m.shape[0]
    return pl.pallas_call(
        _edge_mlp_body,
        grid=(es // blk,),
        in_specs=[
            pl.BlockSpec((blk, D), lambda i: (i, 0)),
            pl.BlockSpec((DE, blk), lambda i: (0, i + blk_off)),
            pl.BlockSpec((DE, D), lambda i: (0, 0)),
            pl.BlockSpec((1, D), lambda i: (0, 0)),
            pl.BlockSpec((D, D), lambda i: (0, 0)),
            pl.BlockSpec((1, D), lambda i: (0, 0)),
        ],
        out_specs=pl.BlockSpec((blk, D), lambda i: (i, 0)),
        out_shape=jax.ShapeDtypeStruct((es, D), _f32),
    )(m, ea_t, w_attr, be1.reshape(1, D), we2, be2.reshape(1, D))


_ZR = 1000


def _make_scatter_body(epw, nchunk):
  def _sc_scatter_body(ef_hbm, row_hbm, zero_hbm, agg_hbm,
                       idx, efv, agg_sh, lsem):
    EPW, NCHUNK = epw, nchunk
    cid = lax.axis_index("c")
    sid = lax.axis_index("s")
    wid = sid * NC + cid

    @pl.when(sid < N // _ZR)
    def _zero():
        pltpu.sync_copy(zero_hbm, agg_sh.at[pl.ds(sid * _ZR, _ZR)])

    plsc.subcore_barrier()

    def fetch(b, i):
        base = wid * EPW + i * CH
        pltpu.sync_copy(row_hbm.at[wid, i], idx[b])
        pltpu.async_copy(ef_hbm.at[pl.ds(base, CH)], efv[b], lsem[b])

    def scat(b, i):
        base = wid * EPW + i * CH
        pltpu.make_async_copy(ef_hbm.at[pl.ds(base, CH)], efv[b],
                              lsem[b]).wait()
        pltpu.sync_copy(efv[b], agg_sh.at[idx[b]], add=True)

    NB = 4
    for b in range(NB):
        fetch(b, b)

    def quad(t, carry):
        i0 = NB * t
        for b in range(NB):
            scat(b, i0 + b)

            @pl.when(i0 + b + NB < NCHUNK)
            def _():
                fetch(b, i0 + b + NB)
        return carry

    lax.fori_loop(0, NCHUNK // NB, quad, 0)
    for r in range(NB * (NCHUNK // NB), NCHUNK):
        scat(r % NB, r)
    plsc.subcore_barrier()

    @pl.when(sid < N // _ZR)
    def _writeback():
        pltpu.sync_copy(agg_sh.at[pl.ds(sid * _ZR, _ZR)],
                        agg_hbm.at[pl.ds(cid * N + sid * _ZR, _ZR)])
  return _sc_scatter_body


def _sc_scatter(ef, row3, zeros_blk, epw):
    mesh = plsc.VectorSubcoreMesh(core_axis_name="c", subcore_axis_name="s")
    f = pl.kernel(
        _make_scatter_body(epw, epw // CH),
        out_type=jax.ShapeDtypeStruct((2 * N, D), _f32),
        mesh=mesh,
        compiler_params=pltpu.CompilerParams(needs_layout_passes=False),
        scratch_types=[
            [pltpu.VMEM((CH,), jnp.int32)] * 4,
            [pltpu.VMEM((CH, D), _f32)] * 4,
            pltpu.VMEM_SHARED((N, D), _f32),
            [pltpu.SemaphoreType.DMA] * 4,
        ],
    )
    return f(ef, row3, zeros_blk)


def _node_body(*refs):
    nagg = len(refs) - 10
    (h_ref, *agg_refs), (b_ref, w1h_ref, w1a_ref, bn1_ref, w2_ref, bn2_ref,
                         o_ref, sums, cnts) = refs[:1 + nagg], refs[1 + nagg:]
    i = pl.program_id(0)

    @pl.when(i == 0)
    def _init():
        sums[...] = jnp.zeros_like(sums)
        cnts[...] = jnp.zeros_like(cnts)

    hb = h_ref[...]
    agg = agg_refs[0][...]
    for a in agg_refs[1:]:
        agg = agg + a[...]
    t = (jnp.dot(hb, w1h_ref[...], preferred_element_type=_f32)
         + jnp.dot(agg, w1a_ref[...], preferred_element_type=_f32)
         + bn1_ref[...])
    t = t * jax.nn.sigmoid(t)
    out = jnp.dot(t, w2_ref[...], preferred_element_type=_f32) + bn2_ref[...]
    hn = hb + out
    R = hb.shape[0]
    oh = jnp.equal(lax.broadcasted_iota(jnp.int32, (128, R), 0),
                   b_ref[0]).astype(_f32)
    sums[...] += jnp.dot(oh, hn, preferred_element_type=_f32)
    cnts[...] += jnp.dot(oh, jnp.ones((R, D), _f32),
                         preferred_element_type=_f32)

    @pl.when(i == pl.num_programs(0) - 1)
    def _fin():
        o_ref[...] = sums[...] / jnp.maximum(cnts[...], 1.0)


def _node_pool(h, aggs, batch3, wn1h, wn1a, bn1, wn2, bn2):
    blk = 1000
    nb = N // blk
    agg_specs = []
    agg_args = []
    for a in aggs:
        agg_specs += [pl.BlockSpec((blk, D), lambda i: (i, 0)),
                      pl.BlockSpec((blk, D), lambda i: (i + nb, 0))]
        agg_args += [a, a]
    return pl.pallas_call(
        _node_body,
        grid=(nb,),
        in_specs=[pl.BlockSpec((blk, D), lambda i: (i, 0))] + agg_specs + [
            pl.BlockSpec((1, 1, blk), lambda i: (i, 0, 0)),
            pl.BlockSpec((D, D), lambda i: (0, 0)),
            pl.BlockSpec((D, D), lambda i: (0, 0)),
            pl.BlockSpec((1, D), lambda i: (0, 0)),
            pl.BlockSpec((D, D), lambda i: (0, 0)),
            pl.BlockSpec((1, D), lambda i: (0, 0)),
        ],
        out_specs=pl.BlockSpec((128, D), lambda i: (0, 0)),
        out_shape=jax.ShapeDtypeStruct((128, D), _f32),
        scratch_shapes=[
            pltpu.VMEM((128, D), _f32),
            pltpu.VMEM((128, D), _f32),
        ],
    )(h, *agg_args, batch3, wn1h, wn1a,
      bn1.reshape(1, D), wn2, bn2.reshape(1, D))


_SLABS = (161280, 158720)


def kernel(h, edge_index, x, edge_attr, batch,
           We1, be1, We2, be2, Wn1, bn1, Wn2, bn2):
    row = edge_index[0]
    col = edge_index[1]

    w_ab = jnp.concatenate([We1[:D], We1[D:2 * D]], axis=1)
    ta, tb = _node_proj(h, w_ab)
    xt = x.T.reshape(3 * N)

    wr = We1[2 * D]
    w_attr = We1[2 * D + 1:]
    ea_t = edge_attr.T
    zeros_blk = jnp.zeros((_ZR, D), _f32)

    aggs = []
    off = 0
    for es in _SLABS:
        epw = es // NW
        row_s = lax.slice(row, (off,), (off + es,))
        col_s = lax.slice(col, (off,), (off + es,))
        m_s = _sc_gather(ta, tb, row_s.reshape(NW, epw),
                         col_s.reshape(NW, epw), wr, xt, epw)
        ef_s = _edge_mlp(m_s, ea_t, w_attr, be1, We2, be2, off // 2560)
        aggs.append(_sc_scatter(ef_s, row_s.reshape(NW, epw // CH, CH),
                                zeros_blk, epw))
        off += es

    batch3 = batch.reshape(N // 1000, 1, 1000)
    p_full = _node_pool(h, aggs, batch3, Wn1[:D], Wn1[D:], bn1, Wn2, bn2)
    return p_full[:G]
